# baseline (speedup 1.0000x reference)
import functools

import jax
import jax.numpy as jnp
from jax import lax
from jax.experimental import pallas as pl
from jax.experimental.pallas import tpu as pltpu
from jax.experimental.pallas import tpu_sc as plsc

CAPACITY_FACTOR = 0.5


def _plan_body(x_ref, wsw_ref, bsw_ref, tfs_ref, gsrc_ref, sel_ref,
               counts_ref, tfs_acc_ref, *, blk, nblk, e, cap):
    i = pl.program_id(0)

    @pl.when(i == 0)
    def _init():
        counts_ref[...] = jnp.zeros_like(counts_ref)
        tfs_acc_ref[...] = jnp.zeros_like(tfs_acc_ref)

    xb = x_ref[...]
    logits = lax.dot_general(
        xb, wsw_ref[...], (((1,), (1,)), ((), ())),
        preferred_element_type=jnp.float32) + bsw_ref[...]
    e_iota = lax.broadcasted_iota(jnp.int32, (blk, e), 1)
    mx = jnp.max(logits, axis=1, keepdims=True)
    routes = jnp.min(jnp.where(logits == mx, e_iota, e), axis=1)
    onehot = (e_iota == routes[:, None]).astype(jnp.float32)

    r_iota = lax.broadcasted_iota(jnp.int32, (blk, blk), 0)
    c_iota = lax.broadcasted_iota(jnp.int32, (blk, blk), 1)
    tril = (r_iota > c_iota).astype(jnp.float32)
    prefix = lax.dot_general(
        tril, onehot, (((1,), (0,)), ((), ())),
        preferred_element_type=jnp.float32)
    posf = jnp.sum(onehot * (prefix + counts_ref[...]), axis=1)
    pos = posf.astype(jnp.int32)
    counts_ref[...] = counts_ref[...] + jnp.sum(onehot, axis=0, keepdims=True)

    kept = pos < cap
    slot = routes * cap + jnp.minimum(pos, cap - 1)
    gsrc_ref[...] = slot.reshape(1, 1, blk)
    sel_ref[...] = jnp.broadcast_to(
        kept.astype(jnp.float32)[:, None], (blk, 16))

    s_iota = lax.broadcasted_iota(jnp.int32, (blk, e * cap), 1)
    ohslot = jnp.where((s_iota == slot[:, None]) & kept[:, None], 1.0, 0.0)
    tvals = ((i * blk) + lax.broadcasted_iota(
        jnp.int32, (1, blk), 1)).astype(jnp.float32)
    tfs_acc_ref[...] += lax.dot_general(
        tvals, ohslot, (((1,), (0,)), ((), ())),
        precision=lax.Precision.HIGHEST,
        preferred_element_type=jnp.float32)

    @pl.when(i == nblk - 1)
    def _fin():
        tfs_ref[...] = tfs_acc_ref[...].astype(jnp.int32)


def _build_plan(n, d, e, cap, blk=256):
    nblk = n // blk
    return pl.pallas_call(
        functools.partial(_plan_body, blk=blk, nblk=nblk, e=e, cap=cap),
        grid=(nblk,),
        in_specs=[
            pl.BlockSpec((blk, d), lambda i: (i, 0)),
            pl.BlockSpec((e, d), lambda i: (0, 0)),
            pl.BlockSpec((1, e), lambda i: (0, 0)),
        ],
        out_specs=[
            pl.BlockSpec((1, e * cap), lambda i: (0, 0)),
            pl.BlockSpec((1, 1, blk), lambda i: (i, 0, 0)),
            pl.BlockSpec((blk, 16), lambda i: (i, 0)),
        ],
        out_shape=[
            jax.ShapeDtypeStruct((1, e * cap), jnp.int32),
            jax.ShapeDtypeStruct((nblk, 1, blk), jnp.int32),
            jax.ShapeDtypeStruct((n, 16), jnp.float32),
        ],
        scratch_shapes=[
            pltpu.VMEM((1, e), jnp.float32),
            pltpu.VMEM((1, e * cap), jnp.float32),
        ],
        compiler_params=pltpu.CompilerParams(
            dimension_semantics=("arbitrary",)),
    )


def _ffn_body(x_ref, w1_ref, b1_ref, w2_ref, b2_ref, y_ref):
    k = pl.program_id(1)
    xb = x_ref[0].astype(jnp.bfloat16)
    h = jnp.maximum(
        lax.dot_general(xb, w1_ref[0].astype(jnp.bfloat16),
                        (((1,), (0,)), ((), ())),
                        preferred_element_type=jnp.float32) + b1_ref[0],
        0.0).astype(jnp.bfloat16)
    contrib = lax.dot_general(
        h, w2_ref[0].astype(jnp.bfloat16), (((1,), (0,)), ((), ())),
        preferred_element_type=jnp.float32)

    @pl.when(k == 0)
    def _first():
        y_ref[0] = contrib + b2_ref[0]

    @pl.when(k != 0)
    def _rest():
        y_ref[0] = y_ref[0] + contrib


def _build_ffn(e, cap, d, dff, fblk=512):
    kblk = dff // fblk
    return pl.pallas_call(
        _ffn_body,
        grid=(e, kblk),
        in_specs=[
            pl.BlockSpec((1, cap, d), lambda ei, k: (ei, 0, 0)),
            pl.BlockSpec((1, d, fblk), lambda ei, k: (ei, 0, k)),
            pl.BlockSpec((1, 1, fblk), lambda ei, k: (ei, 0, k)),
            pl.BlockSpec((1, fblk, d), lambda ei, k: (ei, k, 0)),
            pl.BlockSpec((1, 1, d), lambda ei, k: (ei, 0, 0)),
        ],
        out_specs=pl.BlockSpec((1, cap, d), lambda ei, k: (ei, 0, 0)),
        out_shape=jax.ShapeDtypeStruct((e, cap, d), jnp.float32),
        compiler_params=pltpu.CompilerParams(
            dimension_semantics=("arbitrary", "arbitrary")),
    )


def _build_sc_gather(n_table, n_idx, d):
    info = plsc.get_sparse_core_info()
    nw = info.num_cores * info.num_subcores
    per = n_idx // nw
    mesh = plsc.VectorSubcoreMesh(core_axis_name="c", subcore_axis_name="s")

    @functools.partial(
        pl.kernel,
        out_type=jax.ShapeDtypeStruct((n_idx, d), jnp.float32),
        mesh=mesh,
        scratch_types=[
            pltpu.VMEM((per,), jnp.int32),
            pltpu.VMEM((per, d), jnp.float32),
            pltpu.SemaphoreType.DMA,
        ],
    )
    def gather(table_hbm, idx_hbm, out_hbm, idx_v, rows_v, sem):
        wid = lax.axis_index("s") * info.num_cores + lax.axis_index("c")
        base = wid * per
        pltpu.sync_copy(idx_hbm.at[pl.ds(base, per)], idx_v)
        pltpu.async_copy(table_hbm.at[idx_v], rows_v, sem).wait()
        pltpu.sync_copy(rows_v, out_hbm.at[pl.ds(base, per)])

    return gather


def _build_sc_combine(n, d, chunk=32):
    info = plsc.get_sparse_core_info()
    nw = info.num_cores * info.num_subcores
    per_w = n // nw
    n_chunks = per_w // chunk
    mesh = plsc.VectorSubcoreMesh(core_axis_name="c", subcore_axis_name="s")

    @functools.partial(
        pl.kernel,
        out_type=jax.ShapeDtypeStruct((n, d), jnp.float32),
        mesh=mesh,
        scratch_types=[
            pltpu.VMEM((chunk,), jnp.int32),
            pltpu.VMEM((chunk, 16), jnp.float32),
            pltpu.VMEM((chunk, d), jnp.float32),
            pltpu.VMEM((chunk, d), jnp.float32),
            pltpu.SemaphoreType.DMA,
        ],
    )
    def combine(y_hbm, x_hbm, gsrc_hbm, sel_hbm, out_hbm,
                idx_v, sel_v, y_v, x_v, sem):
        wid = lax.axis_index("s") * info.num_cores + lax.axis_index("c")
        for g in range(n_chunks):
            base = wid * per_w + g * chunk
            pltpu.sync_copy(gsrc_hbm.at[pl.ds(base, chunk)], idx_v)
            pltpu.sync_copy(sel_hbm.at[pl.ds(base, chunk)], sel_v)
            pltpu.async_copy(y_hbm.at[idx_v], y_v, sem).wait()
            pltpu.sync_copy(x_hbm.at[pl.ds(base, chunk)], x_v)

            def row_body(r, carry):
                sv = sel_v[r]
                for c in range(d // 16):
                    xv = x_v[r, pl.ds(c * 16, 16)]
                    yv = y_v[r, pl.ds(c * 16, 16)]
                    x_v[r, pl.ds(c * 16, 16)] = xv + sv * (yv - xv)
                return carry

            lax.fori_loop(0, chunk, row_body, 0)
            pltpu.sync_copy(x_v, out_hbm.at[pl.ds(base, chunk)])

    return combine


def kernel(x, Wsw, bsw, W1, b1, W2, b2):
    b, s, d = x.shape
    n = b * s
    e = Wsw.shape[0]
    dff = W1.shape[2]
    cap = int(CAPACITY_FACTOR * n / e)

    xf = x.reshape(n, d)
    plan = _build_plan(n, d, e, cap)
    tfs2d, gsrc3d, sel = plan(xf, Wsw, bsw.reshape(1, e))

    buf = _build_sc_gather(n, e * cap, d)(xf, tfs2d.reshape(e * cap))
    y = _build_ffn(e, cap, d, dff)(
        buf.reshape(e, cap, d), W1, b1.reshape(e, 1, dff), W2,
        b2.reshape(e, 1, d))
    out = _build_sc_combine(n, d)(
        y.reshape(e * cap, d), xf, gsrc3d.reshape(n), sel)
    return out.reshape(b, s, d)


# speedup vs baseline: 1.4684x; 1.4684x over previous
import functools

import jax
import jax.numpy as jnp
from jax import lax
from jax.experimental import pallas as pl
from jax.experimental.pallas import tpu as pltpu
from jax.experimental.pallas import tpu_sc as plsc

CAPACITY_FACTOR = 0.5


def _plan_body(x_ref, wsw_ref, bsw_ref, tfs_ref, src_ref, ybig_ref,
               counts_ref, tfs_acc_ref, *, blk, nblk, e, cap):
    i = pl.program_id(0)
    n = blk * nblk

    @pl.when(i == 0)
    def _init():
        counts_ref[...] = jnp.zeros_like(counts_ref)
        tfs_acc_ref[...] = jnp.zeros_like(tfs_acc_ref)

    xb = x_ref[...]
    ybig_ref[...] = xb
    logits = lax.dot_general(
        xb, wsw_ref[...], (((1,), (1,)), ((), ())),
        preferred_element_type=jnp.float32) + bsw_ref[...]
    e_iota = lax.broadcasted_iota(jnp.int32, (blk, e), 1)
    mx = jnp.max(logits, axis=1, keepdims=True)
    routes = jnp.min(jnp.where(logits == mx, e_iota, e), axis=1)
    onehot = (e_iota == routes[:, None]).astype(jnp.float32)

    r_iota = lax.broadcasted_iota(jnp.int32, (blk, blk), 0)
    c_iota = lax.broadcasted_iota(jnp.int32, (blk, blk), 1)
    tril = (r_iota > c_iota).astype(jnp.float32)
    prefix = lax.dot_general(
        tril, onehot, (((1,), (0,)), ((), ())),
        preferred_element_type=jnp.float32)
    posf = jnp.sum(onehot * (prefix + counts_ref[...]), axis=1)
    pos = posf.astype(jnp.int32)
    counts_ref[...] = counts_ref[...] + jnp.sum(onehot, axis=0, keepdims=True)

    kept = pos < cap
    slot = routes * cap + jnp.minimum(pos, cap - 1)
    t_ids = i * blk + lax.broadcasted_iota(jnp.int32, (blk,), 0)
    src_ref[...] = jnp.where(kept, n + slot, t_ids).reshape(1, 1, blk)

    s_iota = lax.broadcasted_iota(jnp.int32, (blk, e * cap), 1)
    ohslot = jnp.where((s_iota == slot[:, None]) & kept[:, None], 1.0, 0.0)
    hi = (t_ids // 256).astype(jnp.float32).reshape(1, blk)
    lo = (t_ids % 256).astype(jnp.float32).reshape(1, blk)
    hilo = jnp.concatenate([hi, lo], axis=0)
    tfs_acc_ref[...] += lax.dot_general(
        hilo, ohslot, (((1,), (0,)), ((), ())),
        preferred_element_type=jnp.float32)

    @pl.when(i == nblk - 1)
    def _fin():
        tfs_ref[...] = (256.0 * tfs_acc_ref[0:1] +
                        tfs_acc_ref[1:2]).astype(jnp.int32)


def _build_plan(n, d, e, cap, blk=256):
    nblk = n // blk
    return pl.pallas_call(
        functools.partial(_plan_body, blk=blk, nblk=nblk, e=e, cap=cap),
        grid=(nblk,),
        in_specs=[
            pl.BlockSpec((blk, d), lambda i: (i, 0)),
            pl.BlockSpec((e, d), lambda i: (0, 0)),
            pl.BlockSpec((1, e), lambda i: (0, 0)),
        ],
        out_specs=[
            pl.BlockSpec((1, e * cap), lambda i: (0, 0)),
            pl.BlockSpec((1, 1, blk), lambda i: (i, 0, 0)),
            pl.BlockSpec((blk, d), lambda i: (i, 0)),
        ],
        out_shape=[
            jax.ShapeDtypeStruct((1, e * cap), jnp.int32),
            jax.ShapeDtypeStruct((nblk, 1, blk), jnp.int32),
            jax.ShapeDtypeStruct((n + e * cap, d), jnp.float32),
        ],
        scratch_shapes=[
            pltpu.VMEM((1, e), jnp.float32),
            pltpu.VMEM((2, e * cap), jnp.float32),
        ],
        compiler_params=pltpu.CompilerParams(
            dimension_semantics=("arbitrary",)),
    )


def _ffn_body(x_ref, w1_ref, b1_ref, w2_ref, b2_ref, ybig_in_ref, y_ref):
    del ybig_in_ref
    k = pl.program_id(1)
    xb = x_ref[0].astype(jnp.bfloat16)
    h = jnp.maximum(
        lax.dot_general(xb, w1_ref[0].astype(jnp.bfloat16),
                        (((1,), (0,)), ((), ())),
                        preferred_element_type=jnp.float32) + b1_ref[0],
        0.0).astype(jnp.bfloat16)
    contrib = lax.dot_general(
        h, w2_ref[0].astype(jnp.bfloat16), (((1,), (0,)), ((), ())),
        preferred_element_type=jnp.float32)

    @pl.when(k == 0)
    def _first():
        y_ref[...] = contrib + b2_ref[0]

    @pl.when(k != 0)
    def _rest():
        y_ref[...] = y_ref[...] + contrib


def _build_ffn(n, e, cap, d, dff, fblk=512):
    kblk = dff // fblk
    nblk_off = n // cap
    return pl.pallas_call(
        _ffn_body,
        grid=(e, kblk),
        in_specs=[
            pl.BlockSpec((1, cap, d), lambda ei, k: (ei, 0, 0)),
            pl.BlockSpec((1, d, fblk), lambda ei, k: (ei, 0, k)),
            pl.BlockSpec((1, 1, fblk), lambda ei, k: (ei, 0, k)),
            pl.BlockSpec((1, fblk, d), lambda ei, k: (ei, k, 0)),
            pl.BlockSpec((1, 1, d), lambda ei, k: (ei, 0, 0)),
            pl.BlockSpec(memory_space=pl.ANY),
        ],
        out_specs=pl.BlockSpec((cap, d), lambda ei, k: (nblk_off + ei, 0)),
        out_shape=jax.ShapeDtypeStruct((n + e * cap, d), jnp.float32),
        input_output_aliases={5: 0},
        compiler_params=pltpu.CompilerParams(
            dimension_semantics=("arbitrary", "arbitrary")),
    )


def _build_sc_gather(n_table, n_idx, d, chunk=64):
    info = plsc.get_sparse_core_info()
    nw = info.num_cores * info.num_subcores
    per = n_idx // nw
    n_chunks = per // chunk
    mesh = plsc.VectorSubcoreMesh(core_axis_name="c", subcore_axis_name="s")

    @functools.partial(
        pl.kernel,
        out_type=jax.ShapeDtypeStruct((n_idx, d), jnp.float32),
        mesh=mesh,
        scratch_types=[
            pltpu.VMEM((chunk,), jnp.int32),
            pltpu.VMEM((chunk, d), jnp.float32),
            pltpu.SemaphoreType.DMA,
        ],
    )
    def gather(table_hbm, idx_hbm, out_hbm, idx_v, rows_v, sem):
        wid = lax.axis_index("s") * info.num_cores + lax.axis_index("c")
        for g in range(n_chunks):
            base = wid * per + g * chunk
            pltpu.sync_copy(idx_hbm.at[pl.ds(base, chunk)], idx_v)
            pltpu.async_copy(table_hbm.at[idx_v], rows_v, sem).wait()
            pltpu.sync_copy(rows_v, out_hbm.at[pl.ds(base, chunk)])

    return gather


def kernel(x, Wsw, bsw, W1, b1, W2, b2):
    b, s, d = x.shape
    n = b * s
    e = Wsw.shape[0]
    dff = W1.shape[2]
    cap = int(CAPACITY_FACTOR * n / e)

    xf = x.reshape(n, d)
    tfs2d, src3d, ybig0 = _build_plan(n, d, e, cap)(xf, Wsw, bsw.reshape(1, e))

    buf = _build_sc_gather(n, e * cap, d)(xf, tfs2d.reshape(e * cap))
    ybig = _build_ffn(n, e, cap, d, dff)(
        buf.reshape(e, cap, d), W1, b1.reshape(e, 1, dff), W2,
        b2.reshape(e, 1, d), ybig0)
    out = _build_sc_gather(n + e * cap, n, d)(ybig, src3d.reshape(n))
    return out.reshape(b, s, d)


# speedup vs baseline: 1.5079x; 1.0269x over previous
import functools

import jax
import jax.numpy as jnp
from jax import lax
from jax.experimental import pallas as pl
from jax.experimental.pallas import tpu as pltpu
from jax.experimental.pallas import tpu_sc as plsc

CAPACITY_FACTOR = 0.5


def _plan_body(x_ref, wsw_ref, bsw_ref, tfs_ref, src_ref, ybig_ref,
               counts_ref, tfs_acc_ref, *, blk, nblk, e, cap):
    i = pl.program_id(0)
    n = blk * nblk

    @pl.when(i == 0)
    def _init():
        counts_ref[...] = jnp.zeros_like(counts_ref)
        tfs_acc_ref[...] = jnp.zeros_like(tfs_acc_ref)

    xb = x_ref[...]
    ybig_ref[...] = xb
    logits = lax.dot_general(
        xb, wsw_ref[...], (((1,), (1,)), ((), ())),
        preferred_element_type=jnp.float32) + bsw_ref[...]
    e_iota = lax.broadcasted_iota(jnp.int32, (blk, e), 1)
    mx = jnp.max(logits, axis=1, keepdims=True)
    routes = jnp.min(jnp.where(logits == mx, e_iota, e), axis=1)
    onehot = (e_iota == routes[:, None]).astype(jnp.float32)

    r_iota = lax.broadcasted_iota(jnp.int32, (blk, blk), 0)
    c_iota = lax.broadcasted_iota(jnp.int32, (blk, blk), 1)
    tril = (r_iota > c_iota).astype(jnp.float32)
    prefix = lax.dot_general(
        tril, onehot, (((1,), (0,)), ((), ())),
        preferred_element_type=jnp.float32)
    posf = jnp.sum(onehot * (prefix + counts_ref[...]), axis=1)
    pos = posf.astype(jnp.int32)
    counts_ref[...] = counts_ref[...] + jnp.sum(onehot, axis=0, keepdims=True)

    kept = pos < cap
    slot = routes * cap + jnp.minimum(pos, cap - 1)
    t_ids = i * blk + lax.broadcasted_iota(jnp.int32, (blk,), 0)
    src_ref[...] = jnp.where(kept, n + slot, t_ids).reshape(1, 1, blk)

    p_iota = lax.broadcasted_iota(jnp.int32, (blk, cap), 1)
    ohpos = (p_iota == pos[:, None]).astype(jnp.float32)
    hi = (t_ids // 256).astype(jnp.float32)
    lo = (t_ids % 256).astype(jnp.float32)
    onehot_t = (lax.broadcasted_iota(jnp.int32, (e, blk), 0) ==
                routes[None, :]).astype(jnp.float32)
    lhs = jnp.concatenate(
        [onehot_t * hi[None, :], onehot_t * lo[None, :]], axis=0)
    tfs_acc_ref[...] += lax.dot_general(
        lhs, ohpos, (((1,), (0,)), ((), ())),
        preferred_element_type=jnp.float32)

    @pl.when(i == nblk - 1)
    def _fin():
        tfs_ref[...] = (256.0 * tfs_acc_ref[:e] +
                        tfs_acc_ref[e:]).reshape(1, e * cap).astype(jnp.int32)


def _build_plan(n, d, e, cap, blk=256):
    nblk = n // blk
    return pl.pallas_call(
        functools.partial(_plan_body, blk=blk, nblk=nblk, e=e, cap=cap),
        grid=(nblk,),
        in_specs=[
            pl.BlockSpec((blk, d), lambda i: (i, 0)),
            pl.BlockSpec((e, d), lambda i: (0, 0)),
            pl.BlockSpec((1, e), lambda i: (0, 0)),
        ],
        out_specs=[
            pl.BlockSpec((1, e * cap), lambda i: (0, 0)),
            pl.BlockSpec((1, 1, blk), lambda i: (i, 0, 0)),
            pl.BlockSpec((blk, d), lambda i: (i, 0)),
        ],
        out_shape=[
            jax.ShapeDtypeStruct((1, e * cap), jnp.int32),
            jax.ShapeDtypeStruct((nblk, 1, blk), jnp.int32),
            jax.ShapeDtypeStruct((n + e * cap, d), jnp.float32),
        ],
        scratch_shapes=[
            pltpu.VMEM((1, e), jnp.float32),
            pltpu.VMEM((2 * e, cap), jnp.float32),
        ],
        compiler_params=pltpu.CompilerParams(
            dimension_semantics=("arbitrary",)),
    )


def _ffn_body(x_ref, w1_ref, b1_ref, w2_ref, b2_ref, ybig_in_ref, y_ref):
    del ybig_in_ref
    k = pl.program_id(1)
    xb = x_ref[0].astype(jnp.bfloat16)
    h = jnp.maximum(
        lax.dot_general(xb, w1_ref[0].astype(jnp.bfloat16),
                        (((1,), (0,)), ((), ())),
                        preferred_element_type=jnp.float32) + b1_ref[0],
        0.0).astype(jnp.bfloat16)
    contrib = lax.dot_general(
        h, w2_ref[0].astype(jnp.bfloat16), (((1,), (0,)), ((), ())),
        preferred_element_type=jnp.float32)

    @pl.when(k == 0)
    def _first():
        y_ref[...] = contrib + b2_ref[0]

    @pl.when(k != 0)
    def _rest():
        y_ref[...] = y_ref[...] + contrib


def _build_ffn(n, e, cap, d, dff, fblk=512):
    kblk = dff // fblk
    nblk_off = n // cap
    return pl.pallas_call(
        _ffn_body,
        grid=(e, kblk),
        in_specs=[
            pl.BlockSpec((1, cap, d), lambda ei, k: (ei, 0, 0)),
            pl.BlockSpec((1, d, fblk), lambda ei, k: (ei, 0, k)),
            pl.BlockSpec((1, 1, fblk), lambda ei, k: (ei, 0, k)),
            pl.BlockSpec((1, fblk, d), lambda ei, k: (ei, k, 0)),
            pl.BlockSpec((1, 1, d), lambda ei, k: (ei, 0, 0)),
            pl.BlockSpec(memory_space=pl.ANY),
        ],
        out_specs=pl.BlockSpec((cap, d), lambda ei, k: (nblk_off + ei, 0)),
        out_shape=jax.ShapeDtypeStruct((n + e * cap, d), jnp.float32),
        input_output_aliases={5: 0},
        compiler_params=pltpu.CompilerParams(
            dimension_semantics=("arbitrary", "arbitrary")),
    )


def _build_sc_gather(n_table, n_idx, d, chunk=64):
    info = plsc.get_sparse_core_info()
    nw = info.num_cores * info.num_subcores
    per = n_idx // nw
    n_chunks = per // chunk
    mesh = plsc.VectorSubcoreMesh(core_axis_name="c", subcore_axis_name="s")

    @functools.partial(
        pl.kernel,
        out_type=jax.ShapeDtypeStruct((n_idx, d), jnp.float32),
        mesh=mesh,
        scratch_types=[
            pltpu.VMEM((chunk,), jnp.int32),
            pltpu.VMEM((chunk, d), jnp.float32),
            pltpu.SemaphoreType.DMA,
        ],
    )
    def gather(table_hbm, idx_hbm, out_hbm, idx_v, rows_v, sem):
        wid = lax.axis_index("s") * info.num_cores + lax.axis_index("c")
        for g in range(n_chunks):
            base = wid * per + g * chunk
            pltpu.sync_copy(idx_hbm.at[pl.ds(base, chunk)], idx_v)
            pltpu.async_copy(table_hbm.at[idx_v], rows_v, sem).wait()
            pltpu.sync_copy(rows_v, out_hbm.at[pl.ds(base, chunk)])

    return gather


def kernel(x, Wsw, bsw, W1, b1, W2, b2):
    b, s, d = x.shape
    n = b * s
    e = Wsw.shape[0]
    dff = W1.shape[2]
    cap = int(CAPACITY_FACTOR * n / e)

    xf = x.reshape(n, d)
    tfs2d, src3d, ybig0 = _build_plan(n, d, e, cap)(xf, Wsw, bsw.reshape(1, e))

    buf = _build_sc_gather(n, e * cap, d)(xf, tfs2d.reshape(e * cap))
    ybig = _build_ffn(n, e, cap, d, dff)(
        buf.reshape(e, cap, d), W1, b1.reshape(e, 1, dff), W2,
        b2.reshape(e, 1, d), ybig0)
    out = _build_sc_gather(n + e * cap, n, d)(ybig, src3d.reshape(n))
    return out.reshape(b, s, d)


# speedup vs baseline: 1.6504x; 1.0945x over previous
import functools

import jax
import jax.numpy as jnp
from jax import lax
from jax.experimental import pallas as pl
from jax.experimental.pallas import tpu as pltpu
from jax.experimental.pallas import tpu_sc as plsc

CAPACITY_FACTOR = 0.5


def _plan_body(x_ref, wsw_ref, bsw_ref, tfs_ref, src_ref, ybig_ref,
               counts_ref, tfs_acc_ref, *, blk, nblk, e, cap):
    i = pl.program_id(0)
    n = blk * nblk

    @pl.when(i == 0)
    def _init():
        counts_ref[...] = jnp.zeros_like(counts_ref)
        tfs_acc_ref[...] = jnp.zeros_like(tfs_acc_ref)

    xb = x_ref[...]
    ybig_ref[...] = xb
    logits = lax.dot_general(
        xb, wsw_ref[...], (((1,), (1,)), ((), ())),
        preferred_element_type=jnp.float32) + bsw_ref[...]
    e_iota = lax.broadcasted_iota(jnp.int32, (blk, e), 1)
    mx = jnp.max(logits, axis=1, keepdims=True)
    routes = jnp.min(jnp.where(logits == mx, e_iota, e), axis=1)
    onehot = (e_iota == routes[:, None]).astype(jnp.float32)

    r_iota = lax.broadcasted_iota(jnp.int32, (blk, blk), 0)
    c_iota = lax.broadcasted_iota(jnp.int32, (blk, blk), 1)
    tril = (r_iota > c_iota).astype(jnp.float32)
    prefix = lax.dot_general(
        tril, onehot, (((1,), (0,)), ((), ())),
        preferred_element_type=jnp.float32)
    posf = jnp.sum(onehot * (prefix + counts_ref[...]), axis=1)
    pos = posf.astype(jnp.int32)
    counts_ref[...] = counts_ref[...] + jnp.sum(onehot, axis=0, keepdims=True)

    kept = pos < cap
    slot = routes * cap + jnp.minimum(pos, cap - 1)
    t_ids = i * blk + lax.broadcasted_iota(jnp.int32, (blk,), 0)
    src_ref[...] = jnp.where(kept, n + slot, t_ids).reshape(1, 1, blk)

    p_iota = lax.broadcasted_iota(jnp.int32, (blk, cap), 1)
    ohpos = (p_iota == pos[:, None]).astype(jnp.float32)
    hi = (t_ids // 256).astype(jnp.float32)
    lo = (t_ids % 256).astype(jnp.float32)
    onehot_t = (lax.broadcasted_iota(jnp.int32, (e, blk), 0) ==
                routes[None, :]).astype(jnp.float32)
    lhs = jnp.concatenate(
        [onehot_t * hi[None, :], onehot_t * lo[None, :]], axis=0)
    tfs_acc_ref[...] += lax.dot_general(
        lhs, ohpos, (((1,), (0,)), ((), ())),
        preferred_element_type=jnp.float32)

    @pl.when(i == nblk - 1)
    def _fin():
        tfs_ref[...] = (256.0 * tfs_acc_ref[:e] +
                        tfs_acc_ref[e:]).reshape(1, e * cap).astype(jnp.int32)


def _build_plan(n, d, e, cap, blk=256):
    nblk = n // blk
    return pl.pallas_call(
        functools.partial(_plan_body, blk=blk, nblk=nblk, e=e, cap=cap),
        grid=(nblk,),
        in_specs=[
            pl.BlockSpec((blk, d), lambda i: (i, 0)),
            pl.BlockSpec((e, d), lambda i: (0, 0)),
            pl.BlockSpec((1, e), lambda i: (0, 0)),
        ],
        out_specs=[
            pl.BlockSpec((1, e * cap), lambda i: (0, 0)),
            pl.BlockSpec((1, 1, blk), lambda i: (i, 0, 0)),
            pl.BlockSpec((blk, d), lambda i: (i, 0)),
        ],
        out_shape=[
            jax.ShapeDtypeStruct((1, e * cap), jnp.int32),
            jax.ShapeDtypeStruct((nblk, 1, blk), jnp.int32),
            jax.ShapeDtypeStruct((n + e * cap, d), jnp.float32),
        ],
        scratch_shapes=[
            pltpu.VMEM((1, e), jnp.float32),
            pltpu.VMEM((2 * e, cap), jnp.float32),
        ],
        compiler_params=pltpu.CompilerParams(
            dimension_semantics=("arbitrary",)),
    )


def _ffn_body(x_ref, w1_ref, b1_ref, w2_ref, b2_ref, ybig_in_ref, y_ref):
    del ybig_in_ref
    k = pl.program_id(1)
    xb = x_ref[0].astype(jnp.bfloat16)
    h = jnp.maximum(
        lax.dot_general(xb, w1_ref[0].astype(jnp.bfloat16),
                        (((1,), (0,)), ((), ())),
                        preferred_element_type=jnp.float32) + b1_ref[0],
        0.0).astype(jnp.bfloat16)
    contrib = lax.dot_general(
        h, w2_ref[0].astype(jnp.bfloat16), (((1,), (0,)), ((), ())),
        preferred_element_type=jnp.float32)

    @pl.when(k == 0)
    def _first():
        y_ref[...] = contrib + b2_ref[0]

    @pl.when(k != 0)
    def _rest():
        y_ref[...] = y_ref[...] + contrib


def _build_ffn(n, e, cap, d, dff, fblk=1024):
    kblk = dff // fblk
    nblk_off = n // cap
    return pl.pallas_call(
        _ffn_body,
        grid=(e, kblk),
        in_specs=[
            pl.BlockSpec((1, cap, d), lambda ei, k: (ei, 0, 0)),
            pl.BlockSpec((1, d, fblk), lambda ei, k: (ei, 0, k)),
            pl.BlockSpec((1, 1, fblk), lambda ei, k: (ei, 0, k)),
            pl.BlockSpec((1, fblk, d), lambda ei, k: (ei, k, 0)),
            pl.BlockSpec((1, 1, d), lambda ei, k: (ei, 0, 0)),
            pl.BlockSpec(memory_space=pl.ANY),
        ],
        out_specs=pl.BlockSpec((cap, d), lambda ei, k: (nblk_off + ei, 0)),
        out_shape=jax.ShapeDtypeStruct((n + e * cap, d), jnp.float32),
        input_output_aliases={5: 0},
        compiler_params=pltpu.CompilerParams(
            dimension_semantics=("arbitrary", "arbitrary")),
    )


def _build_sc_gather(n_table, n_idx, d, chunk=64):
    info = plsc.get_sparse_core_info()
    nw = info.num_cores * info.num_subcores
    per = n_idx // nw
    n_chunks = per // chunk
    mesh = plsc.VectorSubcoreMesh(core_axis_name="c", subcore_axis_name="s")

    @functools.partial(
        pl.kernel,
        out_type=jax.ShapeDtypeStruct((n_idx, d), jnp.float32),
        mesh=mesh,
        scratch_types=[
            pltpu.VMEM((chunk,), jnp.int32),
            pltpu.VMEM((chunk, d), jnp.float32),
            pltpu.SemaphoreType.DMA,
        ],
    )
    def gather(table_hbm, idx_hbm, out_hbm, idx_v, rows_v, sem):
        wid = lax.axis_index("s") * info.num_cores + lax.axis_index("c")
        for g in range(n_chunks):
            base = wid * per + g * chunk
            pltpu.sync_copy(idx_hbm.at[pl.ds(base, chunk)], idx_v)
            pltpu.async_copy(table_hbm.at[idx_v], rows_v, sem).wait()
            pltpu.sync_copy(rows_v, out_hbm.at[pl.ds(base, chunk)])

    return gather


def kernel(x, Wsw, bsw, W1, b1, W2, b2):
    b, s, d = x.shape
    n = b * s
    e = Wsw.shape[0]
    dff = W1.shape[2]
    cap = int(CAPACITY_FACTOR * n / e)

    xf = x.reshape(n, d)
    tfs2d, src3d, ybig0 = _build_plan(n, d, e, cap)(xf, Wsw, bsw.reshape(1, e))

    buf = _build_sc_gather(n, e * cap, d)(xf, tfs2d.reshape(e * cap))
    ybig = _build_ffn(n, e, cap, d, dff)(
        buf.reshape(e, cap, d), W1, b1.reshape(e, 1, dff), W2,
        b2.reshape(e, 1, d), ybig0)
    out = _build_sc_gather(n + e * cap, n, d)(ybig, src3d.reshape(n))
    return out.reshape(b, s, d)


# speedup vs baseline: 1.6516x; 1.0007x over previous
import functools

import jax
import jax.numpy as jnp
from jax import lax
from jax.experimental import pallas as pl
from jax.experimental.pallas import tpu as pltpu
from jax.experimental.pallas import tpu_sc as plsc

CAPACITY_FACTOR = 0.5


def _plan_body(x_ref, wsw_ref, bsw_ref, tfs_ref, src_ref, ybig_ref,
               counts_ref, tfs_acc_ref, *, blk, nblk, e, cap):
    i = pl.program_id(0)
    n = blk * nblk

    @pl.when(i == 0)
    def _init():
        counts_ref[...] = jnp.zeros_like(counts_ref)
        tfs_acc_ref[...] = jnp.zeros_like(tfs_acc_ref)

    xb = x_ref[...]
    ybig_ref[...] = xb
    logits = lax.dot_general(
        xb, wsw_ref[...], (((1,), (1,)), ((), ())),
        preferred_element_type=jnp.float32) + bsw_ref[...]
    e_iota = lax.broadcasted_iota(jnp.int32, (blk, e), 1)
    mx = jnp.max(logits, axis=1, keepdims=True)
    routes = jnp.min(jnp.where(logits == mx, e_iota, e), axis=1)
    onehot = (e_iota == routes[:, None]).astype(jnp.float32)

    r_iota = lax.broadcasted_iota(jnp.int32, (blk, blk), 0)
    c_iota = lax.broadcasted_iota(jnp.int32, (blk, blk), 1)
    tril = (r_iota > c_iota).astype(jnp.float32)
    prefix = lax.dot_general(
        tril, onehot, (((1,), (0,)), ((), ())),
        preferred_element_type=jnp.float32)
    posf = jnp.sum(onehot * (prefix + counts_ref[...]), axis=1)
    pos = posf.astype(jnp.int32)
    counts_ref[...] = counts_ref[...] + jnp.sum(onehot, axis=0, keepdims=True)

    kept = pos < cap
    slot = routes * cap + jnp.minimum(pos, cap - 1)
    t_ids = i * blk + lax.broadcasted_iota(jnp.int32, (blk,), 0)
    src_ref[...] = jnp.where(kept, n + slot, t_ids).reshape(1, 1, blk)

    p_iota = lax.broadcasted_iota(jnp.int32, (blk, cap), 1)
    ohpos = (p_iota == pos[:, None]).astype(jnp.float32)
    hi = (t_ids // 256).astype(jnp.float32)
    lo = (t_ids % 256).astype(jnp.float32)
    onehot_t = (lax.broadcasted_iota(jnp.int32, (e, blk), 0) ==
                routes[None, :]).astype(jnp.float32)
    lhs = jnp.concatenate(
        [onehot_t * hi[None, :], onehot_t * lo[None, :]], axis=0)
    tfs_acc_ref[...] += lax.dot_general(
        lhs, ohpos, (((1,), (0,)), ((), ())),
        preferred_element_type=jnp.float32)

    @pl.when(i == nblk - 1)
    def _fin():
        tfs_ref[...] = (256.0 * tfs_acc_ref[:e] +
                        tfs_acc_ref[e:]).reshape(1, e * cap).astype(jnp.int32)


def _build_plan(n, d, e, cap, blk=256):
    nblk = n // blk
    return pl.pallas_call(
        functools.partial(_plan_body, blk=blk, nblk=nblk, e=e, cap=cap),
        grid=(nblk,),
        in_specs=[
            pl.BlockSpec((blk, d), lambda i: (i, 0)),
            pl.BlockSpec((e, d), lambda i: (0, 0)),
            pl.BlockSpec((1, e), lambda i: (0, 0)),
        ],
        out_specs=[
            pl.BlockSpec((1, e * cap), lambda i: (0, 0)),
            pl.BlockSpec((1, 1, blk), lambda i: (i, 0, 0)),
            pl.BlockSpec((blk, d), lambda i: (i, 0)),
        ],
        out_shape=[
            jax.ShapeDtypeStruct((1, e * cap), jnp.int32),
            jax.ShapeDtypeStruct((nblk, 1, blk), jnp.int32),
            jax.ShapeDtypeStruct((n + e * cap, d), jnp.float32),
        ],
        scratch_shapes=[
            pltpu.VMEM((1, e), jnp.float32),
            pltpu.VMEM((2 * e, cap), jnp.float32),
        ],
        compiler_params=pltpu.CompilerParams(
            dimension_semantics=("arbitrary",)),
    )


def _ffn_body(x_ref, w1_ref, b1_ref, w2_ref, b2_ref, ybig_in_ref, y_ref):
    del ybig_in_ref
    k = pl.program_id(1)
    xb = x_ref[0].astype(jnp.bfloat16)
    h = jnp.maximum(
        lax.dot_general(xb, w1_ref[0].astype(jnp.bfloat16),
                        (((1,), (0,)), ((), ())),
                        preferred_element_type=jnp.float32) + b1_ref[0],
        0.0).astype(jnp.bfloat16)
    contrib = lax.dot_general(
        h, w2_ref[0].astype(jnp.bfloat16), (((1,), (0,)), ((), ())),
        preferred_element_type=jnp.float32)

    @pl.when(k == 0)
    def _first():
        y_ref[...] = contrib + b2_ref[0]

    @pl.when(k != 0)
    def _rest():
        y_ref[...] = y_ref[...] + contrib


def _build_ffn(n, e, cap, d, dff, fblk=2048):
    kblk = dff // fblk
    nblk_off = n // cap
    return pl.pallas_call(
        _ffn_body,
        grid=(e, kblk),
        in_specs=[
            pl.BlockSpec((1, cap, d), lambda ei, k: (ei, 0, 0)),
            pl.BlockSpec((1, d, fblk), lambda ei, k: (ei, 0, k)),
            pl.BlockSpec((1, 1, fblk), lambda ei, k: (ei, 0, k)),
            pl.BlockSpec((1, fblk, d), lambda ei, k: (ei, k, 0)),
            pl.BlockSpec((1, 1, d), lambda ei, k: (ei, 0, 0)),
            pl.BlockSpec(memory_space=pl.ANY),
        ],
        out_specs=pl.BlockSpec((cap, d), lambda ei, k: (nblk_off + ei, 0)),
        out_shape=jax.ShapeDtypeStruct((n + e * cap, d), jnp.float32),
        input_output_aliases={5: 0},
        compiler_params=pltpu.CompilerParams(
            dimension_semantics=("arbitrary", "arbitrary")),
    )


def _build_sc_gather(n_table, n_idx, d, chunk=64):
    info = plsc.get_sparse_core_info()
    nw = info.num_cores * info.num_subcores
    per = n_idx // nw
    n_chunks = per // chunk
    mesh = plsc.VectorSubcoreMesh(core_axis_name="c", subcore_axis_name="s")

    @functools.partial(
        pl.kernel,
        out_type=jax.ShapeDtypeStruct((n_idx, d), jnp.float32),
        mesh=mesh,
        scratch_types=[
            pltpu.VMEM((chunk,), jnp.int32),
            pltpu.VMEM((chunk, d), jnp.float32),
            pltpu.SemaphoreType.DMA,
        ],
    )
    def gather(table_hbm, idx_hbm, out_hbm, idx_v, rows_v, sem):
        wid = lax.axis_index("s") * info.num_cores + lax.axis_index("c")
        for g in range(n_chunks):
            base = wid * per + g * chunk
            pltpu.sync_copy(idx_hbm.at[pl.ds(base, chunk)], idx_v)
            pltpu.async_copy(table_hbm.at[idx_v], rows_v, sem).wait()
            pltpu.sync_copy(rows_v, out_hbm.at[pl.ds(base, chunk)])

    return gather


def kernel(x, Wsw, bsw, W1, b1, W2, b2):
    b, s, d = x.shape
    n = b * s
    e = Wsw.shape[0]
    dff = W1.shape[2]
    cap = int(CAPACITY_FACTOR * n / e)

    xf = x.reshape(n, d)
    tfs2d, src3d, ybig0 = _build_plan(n, d, e, cap)(xf, Wsw, bsw.reshape(1, e))

    buf = _build_sc_gather(n, e * cap, d)(xf, tfs2d.reshape(e * cap))
    ybig = _build_ffn(n, e, cap, d, dff)(
        buf.reshape(e, cap, d), W1, b1.reshape(e, 1, dff), W2,
        b2.reshape(e, 1, d), ybig0)
    out = _build_sc_gather(n + e * cap, n, d)(ybig, src3d.reshape(n))
    return out.reshape(b, s, d)


# speedup vs baseline: 1.7393x; 1.0531x over previous
import functools

import jax
import jax.numpy as jnp
from jax import lax
from jax.experimental import pallas as pl
from jax.experimental.pallas import tpu as pltpu
from jax.experimental.pallas import tpu_sc as plsc

CAPACITY_FACTOR = 0.5


def _plan_body(x_ref, wsw_ref, bsw_ref, tfs_ref, src_ref, ybig_ref,
               counts_ref, tfs_acc_ref, *, blk, nblk, e, cap):
    i = pl.program_id(0)
    n = blk * nblk

    @pl.when(i == 0)
    def _init():
        counts_ref[...] = jnp.zeros_like(counts_ref)
        tfs_acc_ref[...] = jnp.zeros_like(tfs_acc_ref)

    xb = x_ref[...]
    ybig_ref[...] = xb
    logits = lax.dot_general(
        xb, wsw_ref[...], (((1,), (1,)), ((), ())),
        preferred_element_type=jnp.float32) + bsw_ref[...]
    e_iota = lax.broadcasted_iota(jnp.int32, (blk, e), 1)
    mx = jnp.max(logits, axis=1, keepdims=True)
    routes = jnp.min(jnp.where(logits == mx, e_iota, e), axis=1)
    onehot = (e_iota == routes[:, None]).astype(jnp.float32)

    r_iota = lax.broadcasted_iota(jnp.int32, (blk, blk), 0)
    c_iota = lax.broadcasted_iota(jnp.int32, (blk, blk), 1)
    tril = (r_iota > c_iota).astype(jnp.float32)
    prefix = lax.dot_general(
        tril, onehot, (((1,), (0,)), ((), ())),
        preferred_element_type=jnp.float32)
    posf = jnp.sum(onehot * (prefix + counts_ref[...]), axis=1)
    pos = posf.astype(jnp.int32)
    counts_ref[...] = counts_ref[...] + jnp.sum(onehot, axis=0, keepdims=True)

    kept = pos < cap
    slot = routes * cap + jnp.minimum(pos, cap - 1)
    t_ids = i * blk + lax.broadcasted_iota(jnp.int32, (blk,), 0)
    src_ref[...] = jnp.where(kept, n + slot, t_ids).reshape(1, 1, blk)

    p_iota = lax.broadcasted_iota(jnp.int32, (blk, cap), 1)
    ohpos = (p_iota == pos[:, None]).astype(jnp.float32)
    hi = (t_ids // 256).astype(jnp.float32)
    lo = (t_ids % 256).astype(jnp.float32)
    onehot_t = (lax.broadcasted_iota(jnp.int32, (e, blk), 0) ==
                routes[None, :]).astype(jnp.float32)
    lhs = jnp.concatenate(
        [onehot_t * hi[None, :], onehot_t * lo[None, :]], axis=0)
    tfs_acc_ref[...] += lax.dot_general(
        lhs, ohpos, (((1,), (0,)), ((), ())),
        preferred_element_type=jnp.float32)

    @pl.when(i == nblk - 1)
    def _fin():
        tfs_ref[...] = (256.0 * tfs_acc_ref[:e] +
                        tfs_acc_ref[e:]).reshape(1, e * cap).astype(jnp.int32)


def _build_plan(n, d, e, cap, blk=512):
    nblk = n // blk
    return pl.pallas_call(
        functools.partial(_plan_body, blk=blk, nblk=nblk, e=e, cap=cap),
        grid=(nblk,),
        in_specs=[
            pl.BlockSpec((blk, d), lambda i: (i, 0)),
            pl.BlockSpec((e, d), lambda i: (0, 0)),
            pl.BlockSpec((1, e), lambda i: (0, 0)),
        ],
        out_specs=[
            pl.BlockSpec((1, e * cap), lambda i: (0, 0)),
            pl.BlockSpec((1, 1, blk), lambda i: (i, 0, 0)),
            pl.BlockSpec((blk, d), lambda i: (i, 0)),
        ],
        out_shape=[
            jax.ShapeDtypeStruct((1, e * cap), jnp.int32),
            jax.ShapeDtypeStruct((nblk, 1, blk), jnp.int32),
            jax.ShapeDtypeStruct((n + e * cap, d), jnp.float32),
        ],
        scratch_shapes=[
            pltpu.VMEM((1, e), jnp.float32),
            pltpu.VMEM((2 * e, cap), jnp.float32),
        ],
        compiler_params=pltpu.CompilerParams(
            dimension_semantics=("arbitrary",)),
    )


def _ffn_body(x_ref, w1_ref, b1_ref, w2_ref, b2_ref, ybig_in_ref, y_ref):
    del ybig_in_ref
    k = pl.program_id(1)
    xb = x_ref[0].astype(jnp.bfloat16)
    h = jnp.maximum(
        lax.dot_general(xb, w1_ref[0].astype(jnp.bfloat16),
                        (((1,), (0,)), ((), ())),
                        preferred_element_type=jnp.float32) + b1_ref[0],
        0.0).astype(jnp.bfloat16)
    contrib = lax.dot_general(
        h, w2_ref[0].astype(jnp.bfloat16), (((1,), (0,)), ((), ())),
        preferred_element_type=jnp.float32)

    @pl.when(k == 0)
    def _first():
        y_ref[...] = contrib + b2_ref[0]

    @pl.when(k != 0)
    def _rest():
        y_ref[...] = y_ref[...] + contrib


def _build_ffn(n, e, cap, d, dff, fblk=1024):
    kblk = dff // fblk
    nblk_off = n // cap
    return pl.pallas_call(
        _ffn_body,
        grid=(e, kblk),
        in_specs=[
            pl.BlockSpec((1, cap, d), lambda ei, k: (ei, 0, 0)),
            pl.BlockSpec((1, d, fblk), lambda ei, k: (ei, 0, k)),
            pl.BlockSpec((1, 1, fblk), lambda ei, k: (ei, 0, k)),
            pl.BlockSpec((1, fblk, d), lambda ei, k: (ei, k, 0)),
            pl.BlockSpec((1, 1, d), lambda ei, k: (ei, 0, 0)),
            pl.BlockSpec(memory_space=pl.ANY),
        ],
        out_specs=pl.BlockSpec((cap, d), lambda ei, k: (nblk_off + ei, 0)),
        out_shape=jax.ShapeDtypeStruct((n + e * cap, d), jnp.float32),
        input_output_aliases={5: 0},
        compiler_params=pltpu.CompilerParams(
            dimension_semantics=("arbitrary", "arbitrary")),
    )


def _build_sc_gather(n_table, n_idx, d, chunk=64):
    info = plsc.get_sparse_core_info()
    nw = info.num_cores * info.num_subcores
    per = n_idx // nw
    n_chunks = per // chunk
    mesh = plsc.VectorSubcoreMesh(core_axis_name="c", subcore_axis_name="s")

    @functools.partial(
        pl.kernel,
        out_type=jax.ShapeDtypeStruct((n_idx, d), jnp.float32),
        mesh=mesh,
        scratch_types=[
            pltpu.VMEM((chunk,), jnp.int32),
            pltpu.VMEM((chunk, d), jnp.float32),
            pltpu.SemaphoreType.DMA,
        ],
    )
    def gather(table_hbm, idx_hbm, out_hbm, idx_v, rows_v, sem):
        wid = lax.axis_index("s") * info.num_cores + lax.axis_index("c")
        for g in range(n_chunks):
            base = wid * per + g * chunk
            pltpu.sync_copy(idx_hbm.at[pl.ds(base, chunk)], idx_v)
            pltpu.async_copy(table_hbm.at[idx_v], rows_v, sem).wait()
            pltpu.sync_copy(rows_v, out_hbm.at[pl.ds(base, chunk)])

    return gather


def kernel(x, Wsw, bsw, W1, b1, W2, b2):
    b, s, d = x.shape
    n = b * s
    e = Wsw.shape[0]
    dff = W1.shape[2]
    cap = int(CAPACITY_FACTOR * n / e)

    xf = x.reshape(n, d)
    tfs2d, src3d, ybig0 = _build_plan(n, d, e, cap)(xf, Wsw, bsw.reshape(1, e))

    buf = _build_sc_gather(n, e * cap, d)(xf, tfs2d.reshape(e * cap))
    ybig = _build_ffn(n, e, cap, d, dff)(
        buf.reshape(e, cap, d), W1, b1.reshape(e, 1, dff), W2,
        b2.reshape(e, 1, d), ybig0)
    out = _build_sc_gather(n + e * cap, n, d)(ybig, src3d.reshape(n))
    return out.reshape(b, s, d)


# speedup vs baseline: 1.7630x; 1.0136x over previous
import functools

import jax
import jax.numpy as jnp
from jax import lax
from jax.experimental import pallas as pl
from jax.experimental.pallas import tpu as pltpu
from jax.experimental.pallas import tpu_sc as plsc

CAPACITY_FACTOR = 0.5


def _plan_body(x_ref, wsw_ref, bsw_ref, tfs_ref, src_ref, ybig_ref,
               counts_ref, tfs_acc_ref, *, blk, nblk, e, cap):
    i = pl.program_id(0)
    n = blk * nblk

    @pl.when(i == 0)
    def _init():
        counts_ref[...] = jnp.zeros_like(counts_ref)
        tfs_acc_ref[...] = jnp.zeros_like(tfs_acc_ref)

    xb = x_ref[...]
    ybig_ref[...] = xb
    logits = lax.dot_general(
        xb, wsw_ref[...], (((1,), (1,)), ((), ())),
        preferred_element_type=jnp.float32) + bsw_ref[...]
    e_iota = lax.broadcasted_iota(jnp.int32, (blk, e), 1)
    mx = jnp.max(logits, axis=1, keepdims=True)
    routes = jnp.min(jnp.where(logits == mx, e_iota, e), axis=1)
    onehot = (e_iota == routes[:, None]).astype(jnp.float32)

    r_iota = lax.broadcasted_iota(jnp.int32, (blk, blk), 0)
    c_iota = lax.broadcasted_iota(jnp.int32, (blk, blk), 1)
    tril = (r_iota > c_iota).astype(jnp.float32)
    prefix = lax.dot_general(
        tril, onehot, (((1,), (0,)), ((), ())),
        preferred_element_type=jnp.float32)
    posf = jnp.sum(onehot * (prefix + counts_ref[...]), axis=1)
    pos = posf.astype(jnp.int32)
    counts_ref[...] = counts_ref[...] + jnp.sum(onehot, axis=0, keepdims=True)

    kept = pos < cap
    slot = routes * cap + jnp.minimum(pos, cap - 1)
    t_ids = i * blk + lax.broadcasted_iota(jnp.int32, (blk,), 0)
    src_ref[...] = jnp.where(kept, n + slot, t_ids).reshape(1, 1, blk)

    p_iota = lax.broadcasted_iota(jnp.int32, (blk, cap), 1)
    ohpos = (p_iota == pos[:, None]).astype(jnp.float32)
    hi = (t_ids // 256).astype(jnp.float32)
    lo = (t_ids % 256).astype(jnp.float32)
    onehot_t = (lax.broadcasted_iota(jnp.int32, (e, blk), 0) ==
                routes[None, :]).astype(jnp.float32)
    lhs = jnp.concatenate(
        [onehot_t * hi[None, :], onehot_t * lo[None, :]], axis=0)
    tfs_acc_ref[...] += lax.dot_general(
        lhs, ohpos, (((1,), (0,)), ((), ())),
        preferred_element_type=jnp.float32)

    @pl.when(i == nblk - 1)
    def _fin():
        tfs_ref[...] = (256.0 * tfs_acc_ref[:e] +
                        tfs_acc_ref[e:]).reshape(1, e * cap).astype(jnp.int32)


def _build_plan(n, d, e, cap, blk=1024):
    nblk = n // blk
    return pl.pallas_call(
        functools.partial(_plan_body, blk=blk, nblk=nblk, e=e, cap=cap),
        grid=(nblk,),
        in_specs=[
            pl.BlockSpec((blk, d), lambda i: (i, 0)),
            pl.BlockSpec((e, d), lambda i: (0, 0)),
            pl.BlockSpec((1, e), lambda i: (0, 0)),
        ],
        out_specs=[
            pl.BlockSpec((1, e * cap), lambda i: (0, 0)),
            pl.BlockSpec((1, 1, blk), lambda i: (i, 0, 0)),
            pl.BlockSpec((blk, d), lambda i: (i, 0)),
        ],
        out_shape=[
            jax.ShapeDtypeStruct((1, e * cap), jnp.int32),
            jax.ShapeDtypeStruct((nblk, 1, blk), jnp.int32),
            jax.ShapeDtypeStruct((n + e * cap, d), jnp.float32),
        ],
        scratch_shapes=[
            pltpu.VMEM((1, e), jnp.float32),
            pltpu.VMEM((2 * e, cap), jnp.float32),
        ],
        compiler_params=pltpu.CompilerParams(
            dimension_semantics=("arbitrary",)),
    )


def _ffn_body(x_ref, w1_ref, b1_ref, w2_ref, b2_ref, ybig_in_ref, y_ref):
    del ybig_in_ref
    k = pl.program_id(1)
    xb = x_ref[0].astype(jnp.bfloat16)
    h = jnp.maximum(
        lax.dot_general(xb, w1_ref[0].astype(jnp.bfloat16),
                        (((1,), (0,)), ((), ())),
                        preferred_element_type=jnp.float32) + b1_ref[0],
        0.0).astype(jnp.bfloat16)
    contrib = lax.dot_general(
        h, w2_ref[0].astype(jnp.bfloat16), (((1,), (0,)), ((), ())),
        preferred_element_type=jnp.float32)

    @pl.when(k == 0)
    def _first():
        y_ref[...] = contrib + b2_ref[0]

    @pl.when(k != 0)
    def _rest():
        y_ref[...] = y_ref[...] + contrib


def _build_ffn(n, e, cap, d, dff, fblk=1024):
    kblk = dff // fblk
    nblk_off = n // cap
    return pl.pallas_call(
        _ffn_body,
        grid=(e, kblk),
        in_specs=[
            pl.BlockSpec((1, cap, d), lambda ei, k: (ei, 0, 0)),
            pl.BlockSpec((1, d, fblk), lambda ei, k: (ei, 0, k)),
            pl.BlockSpec((1, 1, fblk), lambda ei, k: (ei, 0, k)),
            pl.BlockSpec((1, fblk, d), lambda ei, k: (ei, k, 0)),
            pl.BlockSpec((1, 1, d), lambda ei, k: (ei, 0, 0)),
            pl.BlockSpec(memory_space=pl.ANY),
        ],
        out_specs=pl.BlockSpec((cap, d), lambda ei, k: (nblk_off + ei, 0)),
        out_shape=jax.ShapeDtypeStruct((n + e * cap, d), jnp.float32),
        input_output_aliases={5: 0},
        compiler_params=pltpu.CompilerParams(
            dimension_semantics=("arbitrary", "arbitrary")),
    )


def _build_sc_gather(n_table, n_idx, d, chunk=64):
    info = plsc.get_sparse_core_info()
    nw = info.num_cores * info.num_subcores
    per = n_idx // nw
    n_chunks = per // chunk
    mesh = plsc.VectorSubcoreMesh(core_axis_name="c", subcore_axis_name="s")

    @functools.partial(
        pl.kernel,
        out_type=jax.ShapeDtypeStruct((n_idx, d), jnp.float32),
        mesh=mesh,
        scratch_types=[
            pltpu.VMEM((chunk,), jnp.int32),
            pltpu.VMEM((chunk, d), jnp.float32),
            pltpu.SemaphoreType.DMA,
        ],
    )
    def gather(table_hbm, idx_hbm, out_hbm, idx_v, rows_v, sem):
        wid = lax.axis_index("s") * info.num_cores + lax.axis_index("c")
        for g in range(n_chunks):
            base = wid * per + g * chunk
            pltpu.sync_copy(idx_hbm.at[pl.ds(base, chunk)], idx_v)
            pltpu.async_copy(table_hbm.at[idx_v], rows_v, sem).wait()
            pltpu.sync_copy(rows_v, out_hbm.at[pl.ds(base, chunk)])

    return gather


def kernel(x, Wsw, bsw, W1, b1, W2, b2):
    b, s, d = x.shape
    n = b * s
    e = Wsw.shape[0]
    dff = W1.shape[2]
    cap = int(CAPACITY_FACTOR * n / e)

    xf = x.reshape(n, d)
    tfs2d, src3d, ybig0 = _build_plan(n, d, e, cap)(xf, Wsw, bsw.reshape(1, e))

    buf = _build_sc_gather(n, e * cap, d)(xf, tfs2d.reshape(e * cap))
    ybig = _build_ffn(n, e, cap, d, dff)(
        buf.reshape(e, cap, d), W1, b1.reshape(e, 1, dff), W2,
        b2.reshape(e, 1, d), ybig0)
    out = _build_sc_gather(n + e * cap, n, d)(ybig, src3d.reshape(n))
    return out.reshape(b, s, d)
